# baseline (speedup 1.0000x reference)
import functools

import jax
import jax.numpy as jnp
from jax import lax
from jax.experimental import pallas as pl
from jax.experimental.pallas import tpu as pltpu
from jax.experimental.pallas import tpu_sc as plsc

_MAX_LEN = 2048


def kernel(q, k, emb):
    s_q = q.shape[2]
    s_k = k.shape[2]
    d = emb.shape[1]

    emb_pad = jnp.concatenate([emb, emb[-1:]], axis=0)

    info = plsc.get_sparse_core_info()
    nw = info.num_cores * info.num_subcores

    chunk = 64
    nch = s_k // chunk

    mesh = plsc.VectorSubcoreMesh(core_axis_name="c", subcore_axis_name="s")

    @functools.partial(
        pl.kernel,
        mesh=mesh,
        out_type=jax.ShapeDtypeStruct((s_q * s_k * d,), jnp.float32),
    )
    def run(emb_hbm, out_hbm):
        w = lax.axis_index("s") * info.num_cores + lax.axis_index("c")
        src0 = (_MAX_LEN - w) * d
        dst0 = w * s_k * d
        sz = s_k * d
        pltpu.sync_copy(
            emb_hbm.at[pl.ds(src0, sz)],
            out_hbm.at[pl.ds(dst0, sz)],
        )

    out = run(emb_pad.reshape(-1))
    return out.reshape(s_q, s_k, d)


# speedup vs baseline: 15.1657x; 15.1657x over previous
import functools

import jax
import jax.numpy as jnp
from jax import lax
from jax.experimental import pallas as pl
from jax.experimental.pallas import tpu as pltpu
from jax.experimental.pallas import tpu_sc as plsc

_MAX_LEN = 2048


def kernel(q, k, emb):
    s_q = q.shape[2]
    s_k = k.shape[2]
    d = emb.shape[1]

    emb_pad = jnp.concatenate([emb, emb[-1:]], axis=0)

    info = plsc.get_sparse_core_info()
    nw = info.num_cores * info.num_subcores

    chunk = 32
    csz = chunk * d
    nch = s_k // chunk

    mesh = plsc.VectorSubcoreMesh(core_axis_name="c", subcore_axis_name="s")

    @functools.partial(
        pl.kernel,
        mesh=mesh,
        out_type=jax.ShapeDtypeStruct((s_q * s_k * d,), jnp.float32),
        scratch_types=[
            pltpu.VMEM((csz,), jnp.float32),
            pltpu.VMEM((csz,), jnp.float32),
            pltpu.SemaphoreType.DMA,
            pltpu.SemaphoreType.DMA,
            pltpu.SemaphoreType.DMA,
            pltpu.SemaphoreType.DMA,
        ],
    )
    def run(emb_hbm, out_hbm, buf0, buf1, isem0, isem1, osem0, osem1):
        w = lax.axis_index("s") * info.num_cores + lax.axis_index("c")
        src0 = (_MAX_LEN - w) * d
        dst0 = w * s_k * d
        bufs = (buf0, buf1)
        isems = (isem0, isem1)
        osems = (osem0, osem1)

        def in_cp(ci, b):
            return pltpu.make_async_copy(
                emb_hbm.at[pl.ds(src0 + ci * csz, csz)], bufs[b], isems[b])

        def out_cp(ci, b):
            return pltpu.make_async_copy(
                bufs[b], out_hbm.at[pl.ds(dst0 + ci * csz, csz)], osems[b])

        in_cp(0, 0).start()
        in_cp(1, 1).start()

        def body(g, carry):
            ci0 = 2 * g
            for b in range(2):
                ci = ci0 + b
                in_cp(ci, b).wait()
                out_cp(ci, b).start()
                nci = ci + 2

                @pl.when(nci < nch)
                def _():
                    out_cp(ci, b).wait()
                    in_cp(nci, b).start()
            return carry

        lax.fori_loop(0, nch // 2, body, 0)
        out_cp(nch - 2, 0).wait()
        out_cp(nch - 1, 1).wait()

    out = run(emb_pad.reshape(-1))
    return out.reshape(s_q, s_k, d)
